# baseline (speedup 1.0000x reference)
import jax
import jax.numpy as jnp
from jax import lax
from jax.experimental import pallas as pl
from jax.experimental.pallas import tpu as pltpu
from jax.experimental.pallas import tpu_sc as plsc

NUM_CLASS = 100000
N_CLS_CTX = 4
CTX_DIM = 512
N_PRE = 5
N_SUF = 68
SEQ = 77
BATCH = 4096

_NC = 2
_NS = 16
_NW = _NC * _NS
_BPW = BATCH // _NW


def _sc_body(idx_hbm, table_hbm, prefix_hbm, suffix_hbm, out_hbm,
             idx_v, tmpl, gsem):
    wid = lax.axis_index("s") * _NC + lax.axis_index("c")
    base = wid * _BPW

    pltpu.sync_copy(idx_hbm.at[pl.ds(base * 8, _BPW * 8)], idx_v)
    pltpu.sync_copy(prefix_hbm.at[0], tmpl.at[pl.ds(0, N_PRE)])
    pltpu.sync_copy(suffix_hbm.at[0], tmpl.at[pl.ds(N_PRE + N_CLS_CTX, N_SUF)])

    def step(b, carry):
        pltpu.async_copy(table_hbm.at[idx_v.at[pl.ds(8 * b, N_CLS_CTX)]],
                         tmpl.at[pl.ds(N_PRE, N_CLS_CTX)], gsem).wait()
        pltpu.sync_copy(tmpl, out_hbm.at[base + b])
        return carry

    lax.fori_loop(0, _BPW, step, 0)


def kernel(label, cls_ctx, token_prefix, token_suffix):
    table = cls_ctx.reshape(NUM_CLASS * N_CLS_CTX, CTX_DIM)
    pat = jnp.arange(8, dtype=jnp.int32) & 3
    idx = (label[:, None] * N_CLS_CTX + pat[None, :]).reshape(-1)
    mesh = plsc.VectorSubcoreMesh(core_axis_name="c", subcore_axis_name="s")
    f = pl.kernel(
        _sc_body,
        out_type=jax.ShapeDtypeStruct((BATCH, SEQ, CTX_DIM), jnp.float32),
        mesh=mesh,
        compiler_params=pltpu.CompilerParams(use_tc_tiling_on_sc=False),
        scratch_types=[
            pltpu.VMEM((_BPW * 8,), jnp.int32),
            pltpu.VMEM((SEQ, CTX_DIM), jnp.float32),
            pltpu.SemaphoreType.DMA,
        ],
    )
    return f(idx, table, token_prefix, token_suffix)


# speedup vs baseline: 1.0463x; 1.0463x over previous
import jax
import jax.numpy as jnp
from jax import lax
from jax.experimental import pallas as pl
from jax.experimental.pallas import tpu as pltpu
from jax.experimental.pallas import tpu_sc as plsc

NUM_CLASS = 100000
N_CLS_CTX = 4
CTX_DIM = 512
N_PRE = 5
N_SUF = 68
SEQ = 77
BATCH = 4096

_NC = 2
_NS = 16
_NW = _NC * _NS
_BPW = BATCH // _NW
_PHASES = 8
_RPP = _BPW // _PHASES
_GR = _RPP * N_CLS_CTX
_CLEN = N_SUF + N_PRE


def _sc_body(gidx_hbm, didx_hbm, table_hbm, tmplc_hbm, out_hbm,
             gidx_v, didx_v, tmplc_v, gbuf, gsem, ssem, wsem):
    wid = lax.axis_index("s") * _NC + lax.axis_index("c")
    rbase = wid * (_BPW * SEQ)

    pltpu.sync_copy(gidx_hbm.at[pl.ds(wid * _PHASES, _PHASES)], gidx_v)
    pltpu.sync_copy(didx_hbm.at[pl.ds(wid * _PHASES, _PHASES)], didx_v)
    pltpu.sync_copy(tmplc_hbm, tmplc_v)

    pltpu.async_copy(tmplc_v.at[pl.ds(N_SUF, N_PRE)],
                     out_hbm.at[pl.ds(rbase, N_PRE)], wsem)

    sc_h = {}
    for p in range(_PHASES):
        pb = p % 2
        if p >= 2:
            sc_h.pop(p - 2).wait()
        g_h = pltpu.async_copy(table_hbm.at[gidx_v.at[p]], gbuf.at[pb], gsem)

        nfull = _RPP if p < _PHASES - 1 else _RPP - 1

        def cwrite(i, carry, p=p):
            r = rbase + (p * _RPP + i) * SEQ + N_PRE + N_CLS_CTX
            pltpu.async_copy(tmplc_v, out_hbm.at[pl.ds(r, _CLEN)], wsem)
            return carry

        lax.fori_loop(0, nfull, cwrite, 0)

        g_h.wait()
        sc_h[p] = pltpu.async_copy(gbuf.at[pb], out_hbm.at[didx_v.at[p]], ssem)

    pltpu.async_copy(
        tmplc_v.at[pl.ds(0, N_SUF)],
        out_hbm.at[pl.ds(rbase + (_BPW - 1) * SEQ + N_PRE + N_CLS_CTX, N_SUF)],
        wsem)

    sc_h.pop(_PHASES - 2).wait()
    sc_h.pop(_PHASES - 1).wait()

    pltpu.make_async_copy(tmplc_v.at[pl.ds(N_SUF, N_PRE)],
                          out_hbm.at[pl.ds(0, N_PRE)], wsem).wait()
    pltpu.make_async_copy(tmplc_v.at[pl.ds(0, N_SUF)],
                          out_hbm.at[pl.ds(0, N_SUF)], wsem).wait()

    def dwait(i, carry):
        pltpu.make_async_copy(tmplc_v, out_hbm.at[pl.ds(0, _CLEN)], wsem).wait()
        return carry

    lax.fori_loop(0, _BPW - 1, dwait, 0)


def kernel(label, cls_ctx, token_prefix, token_suffix):
    table = cls_ctx.reshape(NUM_CLASS * N_CLS_CTX, CTX_DIM)
    j4 = jnp.arange(N_CLS_CTX, dtype=jnp.int32)
    lab = label.astype(jnp.int32).reshape(_NW * _PHASES, _RPP)
    gidx = (lab[:, :, None] * N_CLS_CTX + j4).reshape(_NW * _PHASES, _GR)
    bglob = jnp.arange(BATCH, dtype=jnp.int32).reshape(_NW * _PHASES, _RPP)
    didx = (bglob[:, :, None] * SEQ + N_PRE + j4).reshape(_NW * _PHASES, _GR)
    tmplc = jnp.concatenate([token_suffix[0], token_prefix[0]], axis=0)

    mesh = plsc.VectorSubcoreMesh(core_axis_name="c", subcore_axis_name="s")
    f = pl.kernel(
        _sc_body,
        out_type=jax.ShapeDtypeStruct((BATCH * SEQ, CTX_DIM), jnp.float32),
        mesh=mesh,
        compiler_params=pltpu.CompilerParams(use_tc_tiling_on_sc=False),
        scratch_types=[
            pltpu.VMEM((_PHASES, _GR), jnp.int32),
            pltpu.VMEM((_PHASES, _GR), jnp.int32),
            pltpu.VMEM((_CLEN, CTX_DIM), jnp.float32),
            pltpu.VMEM((2, _GR, CTX_DIM), jnp.float32),
            pltpu.SemaphoreType.DMA,
            pltpu.SemaphoreType.DMA,
            pltpu.SemaphoreType.DMA,
        ],
    )
    out = f(gidx, didx, table, tmplc)
    return out.reshape(BATCH, SEQ, CTX_DIM)
